# baseline (speedup 1.0000x reference)
import jax
import jax.numpy as jnp
import numpy as np
from jax.experimental import pallas as pl
import jax.experimental.pallas.tpu as pltpu
from jax.sharding import Mesh, PartitionSpec as P

try:
    from jax.experimental.shard_map import shard_map
except ImportError:
    from jax import shard_map

TOKENS = 8192
D_IN = 2048
D_OUT = 2048
BM = 1024


def _matmul_kernel(x_ref, w_ref, o_ref, w_bf16_ref):
    @pl.when(pl.program_id(0) == 0)
    def _():
        w_bf16_ref[...] = w_ref[...].astype(jnp.bfloat16)

    x_bf16 = x_ref[...].astype(jnp.bfloat16)
    o_ref[...] = jax.lax.dot_general(
        x_bf16,
        w_bf16_ref[...],
        dimension_numbers=(((1,), (1,)), ((), ())),
        preferred_element_type=jnp.float32,
    )


def _gemm(x, W):
    m = x.shape[0]
    return pl.pallas_call(
        _matmul_kernel,
        grid=(m // BM,),
        in_specs=[
            pl.BlockSpec((BM, D_IN), lambda i: (i, 0)),
            pl.BlockSpec((D_OUT, D_IN), lambda i: (0, 0)),
        ],
        out_specs=pl.BlockSpec((BM, D_OUT), lambda i: (i, 0)),
        out_shape=jax.ShapeDtypeStruct((m, D_OUT), jnp.float32),
        scratch_shapes=[pltpu.VMEM((D_OUT, D_IN), jnp.bfloat16)],
        compiler_params=pltpu.CompilerParams(
            vmem_limit_bytes=62 * 1024 * 1024,
        ),
    )(x, W)


@jax.jit
def kernel(x, W):
    devs = jax.devices()
    if len(devs) >= 2:
        mesh = Mesh(np.array(devs[:2]), ("m",))
        f = shard_map(
            _gemm,
            mesh=mesh,
            in_specs=(P("m", None), P(None, None)),
            out_specs=P("m", None),
            check_rep=False,
        )
        return f(x, W)
    return _gemm(x, W)


# speedup vs baseline: 3.5549x; 3.5549x over previous
import jax
import jax.numpy as jnp
from jax.experimental import pallas as pl
import jax.experimental.pallas.tpu as pltpu

TOKENS = 8192
D_IN = 2048
D_OUT = 2048
BM = 1024
KC = 512
NM = TOKENS // BM
NK = D_IN // KC


def _matmul_kernel(x_ref, w_hbm_ref, o_ref, w_bf16_ref, stage_ref, sem_ref):
    k = pl.program_id(1)
    slot = jax.lax.rem(k, 2)

    @pl.when(pl.program_id(0) == 0)
    def _():
        @pl.when(k == 0)
        def _():
            for c in (0, 1):
                pltpu.make_async_copy(
                    w_hbm_ref.at[:, pl.ds(c * KC, KC)],
                    stage_ref.at[c],
                    sem_ref.at[c],
                ).start()

        pltpu.make_async_copy(
            w_hbm_ref.at[:, pl.ds(k * KC, KC)],
            stage_ref.at[slot],
            sem_ref.at[slot],
        ).wait()
        w_bf16_ref[k] = stage_ref[slot].astype(jnp.bfloat16)

        @pl.when(k + 2 < NK)
        def _():
            pltpu.make_async_copy(
                w_hbm_ref.at[:, pl.ds((k + 2) * KC, KC)],
                stage_ref.at[slot],
                sem_ref.at[slot],
            ).start()

    x_bf16 = x_ref[...].astype(jnp.bfloat16)
    partial = jax.lax.dot_general(
        x_bf16,
        w_bf16_ref[k],
        dimension_numbers=(((1,), (1,)), ((), ())),
        preferred_element_type=jnp.float32,
    )

    @pl.when(k == 0)
    def _():
        o_ref[...] = partial

    @pl.when(k > 0)
    def _():
        o_ref[...] += partial


@jax.jit
def kernel(x, W):
    return pl.pallas_call(
        _matmul_kernel,
        grid=(NM, NK),
        in_specs=[
            pl.BlockSpec((BM, KC), lambda m, k: (m, k)),
            pl.BlockSpec(memory_space=pl.ANY),
        ],
        out_specs=pl.BlockSpec((BM, D_OUT), lambda m, k: (m, 0)),
        out_shape=jax.ShapeDtypeStruct((TOKENS, D_OUT), jnp.float32),
        scratch_shapes=[
            pltpu.VMEM((NK, D_OUT, KC), jnp.bfloat16),
            pltpu.VMEM((2, D_OUT, KC), jnp.float32),
            pltpu.SemaphoreType.DMA((2,)),
        ],
        compiler_params=pltpu.CompilerParams(
            vmem_limit_bytes=62 * 1024 * 1024,
        ),
    )(x, W)


# speedup vs baseline: 4.8973x; 1.3776x over previous
import jax
import jax.numpy as jnp
from jax.experimental import pallas as pl
import jax.experimental.pallas.tpu as pltpu

TOKENS = 8192
D_IN = 2048
D_OUT = 2048
BM = 1024


def _matmul_kernel(x_ref, w_ref, o_ref, w_bf16_ref):
    @pl.when(pl.program_id(0) == 0)
    def _():
        w_bf16_ref[...] = w_ref[...].astype(jnp.bfloat16)

    x_bf16 = x_ref[...].astype(jnp.bfloat16)
    o_ref[...] = jax.lax.dot_general(
        x_bf16,
        w_bf16_ref[...],
        dimension_numbers=(((1,), (1,)), ((), ())),
        preferred_element_type=jnp.float32,
    )


@jax.jit
def kernel(x, W):
    return pl.pallas_call(
        _matmul_kernel,
        grid=(TOKENS // BM,),
        in_specs=[
            pl.BlockSpec((BM, D_IN), lambda i: (i, 0)),
            pl.BlockSpec((D_OUT, D_IN), lambda i: (0, 0)),
        ],
        out_specs=pl.BlockSpec((BM, D_OUT), lambda i: (i, 0)),
        out_shape=jax.ShapeDtypeStruct((TOKENS, D_OUT), jnp.float32),
        scratch_shapes=[pltpu.VMEM((D_OUT, D_IN), jnp.bfloat16)],
        compiler_params=pltpu.CompilerParams(
            vmem_limit_bytes=62 * 1024 * 1024,
        ),
    )(x, W)
